# baseline (speedup 1.0000x reference)
import functools

import jax
import jax.numpy as jnp
from jax import lax
from jax.experimental import pallas as pl
from jax.experimental.pallas import tpu as pltpu
from jax.experimental.pallas import tpu_sc as plsc

_NUM_CORES = 2
_NUM_SUBCORES = 16
_NW = _NUM_CORES * _NUM_SUBCORES
_CHUNK = 128
_BLK = 2048
_HALVES = 2


def _sc_gather(emb, nbr_idx, center_idx=None):
    t_total = nbr_idx.shape[0]
    d = emb.shape[1]
    per_w = t_total // _NW
    n_chunks = per_w // _CHUNK
    idx3 = nbr_idx.reshape(_NW, n_chunks, _CHUNK)
    do_ctr = center_idx is not None
    n_ctr = center_idx.shape[0] if do_ctr else 0

    mesh = plsc.VectorSubcoreMesh(core_axis_name="c", subcore_axis_name="s",
                                  num_cores=_NUM_CORES,
                                  num_subcores=_NUM_SUBCORES)

    nbuf = min(4, n_chunks)
    out_type = [jax.ShapeDtypeStruct((t_total, d), jnp.float32)]
    scratch = (
        [pltpu.VMEM((n_chunks, _CHUNK), jnp.int32)]
        + [pltpu.VMEM((_CHUNK, d), jnp.float32) for _ in range(nbuf)]
        + [pltpu.SemaphoreType.DMA for _ in range(2 * nbuf)]
    )
    if do_ctr:
        out_type.append(jax.ShapeDtypeStruct((n_ctr, d), jnp.float32))
        scratch = scratch + [
            pltpu.VMEM((n_ctr,), jnp.int32),
            pltpu.VMEM((n_ctr, d), jnp.float32),
            pltpu.SemaphoreType.DMA,
        ]

    @functools.partial(
        pl.kernel, mesh=mesh, out_type=tuple(out_type),
        scratch_types=scratch,
    )
    def gather_kernel(emb_hbm, idx_hbm, *rest):
        if do_ctr:
            ctr_idx_hbm = rest[0]
            out_hbm, ctr_out_hbm = rest[1], rest[2]
            rest = rest[3:]
        else:
            out_hbm = rest[0]
            rest = rest[1:]
        idx_v = rest[0]
        bufs = rest[1:1 + nbuf]
        gsems = rest[1 + nbuf:1 + 2 * nbuf]
        osems = rest[1 + 2 * nbuf:1 + 3 * nbuf]
        if do_ctr:
            ctr_idx_v, ctr_rows, sem_c = rest[1 + 3 * nbuf:]
        wid = lax.axis_index("s") * _NUM_CORES + lax.axis_index("c")
        base = wid * per_w

        pltpu.sync_copy(idx_hbm.at[wid], idx_v)
        gh = [None] * n_chunks
        oh = [None] * n_chunks
        for j in range(nbuf):
            gh[j] = pltpu.async_copy(
                emb_hbm.at[idx_v.at[j]], bufs[j], gsems[j])
        for j in range(n_chunks):
            s = j % nbuf
            gh[j].wait()
            oh[j] = pltpu.async_copy(
                bufs[s], out_hbm.at[pl.ds(base + j * _CHUNK, _CHUNK)],
                osems[s])
            nxt = j + nbuf
            if nxt < n_chunks:
                oh[j].wait()
                gh[nxt] = pltpu.async_copy(
                    emb_hbm.at[idx_v.at[nxt]], bufs[s], gsems[s])
        for j in range(max(0, n_chunks - nbuf), n_chunks):
            oh[j].wait()

        if do_ctr:
            @pl.when(wid == 0)
            def _():
                pltpu.sync_copy(ctr_idx_hbm, ctr_idx_v)
                pltpu.async_copy(emb_hbm.at[ctr_idx_v], ctr_rows, sem_c).wait()
                pltpu.sync_copy(ctr_rows, ctr_out_hbm)

    if do_ctr:
        return gather_kernel(emb, idx3, center_idx)
    res = gather_kernel(emb, idx3)
    return res[0] if isinstance(res, (tuple, list)) else res


def _dg(a, b, dims):
    return lax.dot_general(a, b, (dims, ((), ())),
                           preferred_element_type=jnp.float32)


def _tc_mlp(gathered, ctr_rows, ef32, seg_cols,
            w1a2a, w1a2b, w1b2a, w1b2b, w1cab, b1m, w2two, b2r, p0):
    t_half, d = gathered.shape
    n_ctr = ctr_rows.shape[0]
    blk, nb_total = seg_cols.shape
    nb_h = t_half // blk
    nsteps = nb_h // 2

    def body(g_ref, c_ref, ef_ref, seg_ref,
             w1a2a_ref, w1a2b_ref, w1b2a_ref, w1b2b_ref,
             w1cab_ref, b1m_ref, w2two_ref, b2_ref, out_ref):
        i = pl.program_id(0)
        sel32 = (lax.broadcasted_iota(jnp.int32, (nb_total, 2 * n_ctr), 0) ==
                 p0 + 2 * i +
                 (lax.broadcasted_iota(jnp.int32, (nb_total, 2 * n_ctr), 1)
                  // n_ctr)).astype(jnp.float32)
        segsel = _dg(seg_ref[...], sel32, (((1,), (0,))))
        modrow = (lax.broadcasted_iota(jnp.int32, (1, 2 * n_ctr), 1)
                  % n_ctr).astype(jnp.float32)
        oh2 = (segsel == modrow).astype(jnp.float32)

        ctr = c_ref[...]
        cmab = jnp.concatenate([
            _dg(ctr, w1a2a_ref[...], (((1,), (0,)))),
            _dg(ctr, w1a2b_ref[...], (((1,), (0,)))),
        ], axis=0) + b1m_ref[...]

        g = g_ref[...]
        ef2 = ef_ref[0]
        pre = _dg(g[:blk], w1b2a_ref[...], (((1,), (0,))))
        pre += _dg(g[blk:], w1b2b_ref[...], (((1,), (0,))))
        pre += _dg(ef2, w1cab_ref[...], (((0,), (0,))))
        pre += _dg(oh2, cmab, (((1,), (0,))))
        h = jnp.maximum(pre, 0.0)
        pair = _dg(h, w2two_ref[...], (((1,), (0,))))
        scat16 = (lax.broadcasted_iota(jnp.int32, (2, nb_h), 1) ==
                  2 * i + lax.broadcasted_iota(jnp.int32, (2, nb_h), 0)
                  ).astype(jnp.float32)
        scat = _dg(pair, scat16, (((1,), (0,))))

        @pl.when(i == 0)
        def _():
            out_ref[...] = scat + b2_ref[...]

        @pl.when(i > 0)
        def _():
            out_ref[...] += scat

    pair0 = p0 // 2
    return pl.pallas_call(
        body,
        grid=(nsteps,),
        in_specs=[
            pl.BlockSpec((2 * blk, d), lambda i: (i, 0)),
            pl.BlockSpec((n_ctr, d), lambda i: (0, 0)),
            pl.BlockSpec((1, 2 * 16, blk), lambda i, o=pair0: (i + o, 0, 0)),
            pl.BlockSpec((blk, nb_total), lambda i: (0, 0)),
            pl.BlockSpec((d, 2 * 64), lambda i: (0, 0)),
            pl.BlockSpec((d, 2 * 64), lambda i: (0, 0)),
            pl.BlockSpec((d, 2 * 64), lambda i: (0, 0)),
            pl.BlockSpec((d, 2 * 64), lambda i: (0, 0)),
            pl.BlockSpec((2 * 16, 2 * 64), lambda i: (0, 0)),
            pl.BlockSpec((2 * n_ctr, 2 * 64), lambda i: (0, 0)),
            pl.BlockSpec((2 * 64, 2), lambda i: (0, 0)),
            pl.BlockSpec((1, 1), lambda i: (0, 0)),
        ],
        out_specs=pl.BlockSpec((blk, nb_h), lambda i: (0, 0)),
        out_shape=jax.ShapeDtypeStruct((blk, nb_h), jnp.float32),
    )(gathered, ctr_rows, ef32, seg_cols,
      w1a2a, w1a2b, w1b2a, w1b2b, w1cab, b1m, w2two, b2r)


def kernel(emb, center_idx, neighbor_idx, edge_feats, segment_ids,
           W1, b1, W2, b2):
    center_idx = center_idx.astype(jnp.int32)
    t_total = neighbor_idx.shape[0]
    d = emb.shape[1]
    d_mid = W1.shape[1]
    nb = t_total // _BLK
    t_half = t_total // _HALVES
    nb_h = nb // _HALVES

    w1a, w1b, w1c = W1[:d], W1[d:2 * d], W1[2 * d:]
    z = jnp.zeros_like
    pad_r = lambda w: jnp.concatenate([w, z(w)], axis=1)
    pad_l = lambda w: jnp.concatenate([z(w), w], axis=1)
    w2two = jnp.zeros((2 * d_mid, 2), jnp.float32)
    w2two = w2two.at[:d_mid, 0].set(W2[:, 0]).at[d_mid:, 1].set(W2[:, 0])
    w1cab = jnp.concatenate([pad_r(w1c), pad_l(w1c)], axis=0)
    b1a = jnp.concatenate([b1, jnp.zeros_like(b1)]).reshape(1, 2 * d_mid)
    b1b = jnp.concatenate([jnp.zeros_like(b1), b1]).reshape(1, 2 * d_mid)
    n_ctr = center_idx.shape[0]
    b1m = jnp.concatenate([jnp.broadcast_to(b1a, (n_ctr, 2 * d_mid)),
                           jnp.broadcast_to(b1b, (n_ctr, 2 * d_mid))], axis=0)
    seg_cols = segment_ids.reshape(nb, _BLK).T.astype(jnp.float32)
    b2r = b2.reshape(1, 1)

    ef_t = edge_feats.T
    d_e = edge_feats.shape[1]
    s_total = t_total // (2 * _BLK)
    ef32 = (ef_t.reshape(d_e, s_total, 2, _BLK)
            .transpose(1, 2, 0, 3).reshape(s_total, 2 * d_e, _BLK))

    gathered = []
    ctr_rows = None
    for h in range(_HALVES):
        sl = neighbor_idx[h * t_half:(h + 1) * t_half]
        if h == 0:
            g0, ctr_rows = _sc_gather(emb, sl, center_idx)
            gathered.append(g0)
        else:
            gathered.append(_sc_gather(emb, sl))

    outs = []
    for h in range(_HALVES):
        outs.append(_tc_mlp(
            gathered[h], ctr_rows, ef32, seg_cols,
            pad_r(w1a), pad_l(w1a), pad_r(w1b), pad_l(w1b),
            w1cab, b1m, w2two, b2r,
            p0=h * nb_h))
    out = outs[0] if _HALVES == 1 else jnp.concatenate(outs, axis=1)
    return out.T.reshape(t_total)


# speedup vs baseline: 1.0536x; 1.0536x over previous
import functools

import jax
import jax.numpy as jnp
from jax import lax
from jax.experimental import pallas as pl
from jax.experimental.pallas import tpu as pltpu
from jax.experimental.pallas import tpu_sc as plsc

_NUM_CORES = 2
_NUM_SUBCORES = 16
_NW = _NUM_CORES * _NUM_SUBCORES
_CHUNK = 128
_BLK = 2048
_HALVES = 1


def _sc_gather(emb, nbr_idx, center_idx=None):
    t_total = nbr_idx.shape[0]
    d = emb.shape[1]
    per_w = t_total // _NW
    n_chunks = per_w // _CHUNK
    idx3 = nbr_idx.reshape(_NW, n_chunks, _CHUNK)
    do_ctr = center_idx is not None
    n_ctr = center_idx.shape[0] if do_ctr else 0

    mesh = plsc.VectorSubcoreMesh(core_axis_name="c", subcore_axis_name="s",
                                  num_cores=_NUM_CORES,
                                  num_subcores=_NUM_SUBCORES)

    nbuf = min(4, n_chunks)
    out_type = [jax.ShapeDtypeStruct((t_total, d), jnp.float32)]
    scratch = (
        [pltpu.VMEM((n_chunks, _CHUNK), jnp.int32)]
        + [pltpu.VMEM((_CHUNK, d), jnp.float32) for _ in range(nbuf)]
        + [pltpu.SemaphoreType.DMA for _ in range(2 * nbuf)]
    )
    if do_ctr:
        out_type.append(jax.ShapeDtypeStruct((n_ctr, d), jnp.float32))
        scratch = scratch + [
            pltpu.VMEM((n_ctr,), jnp.int32),
            pltpu.VMEM((n_ctr, d), jnp.float32),
            pltpu.SemaphoreType.DMA,
        ]

    @functools.partial(
        pl.kernel, mesh=mesh, out_type=tuple(out_type),
        scratch_types=scratch,
    )
    def gather_kernel(emb_hbm, idx_hbm, *rest):
        if do_ctr:
            ctr_idx_hbm = rest[0]
            out_hbm, ctr_out_hbm = rest[1], rest[2]
            rest = rest[3:]
        else:
            out_hbm = rest[0]
            rest = rest[1:]
        idx_v = rest[0]
        bufs = rest[1:1 + nbuf]
        gsems = rest[1 + nbuf:1 + 2 * nbuf]
        osems = rest[1 + 2 * nbuf:1 + 3 * nbuf]
        if do_ctr:
            ctr_idx_v, ctr_rows, sem_c = rest[1 + 3 * nbuf:]
        wid = lax.axis_index("s") * _NUM_CORES + lax.axis_index("c")
        base = wid * per_w

        pltpu.sync_copy(idx_hbm.at[wid], idx_v)
        gh = [None] * n_chunks
        oh = [None] * n_chunks
        for j in range(nbuf):
            gh[j] = pltpu.async_copy(
                emb_hbm.at[idx_v.at[j]], bufs[j], gsems[j])
        for j in range(n_chunks):
            s = j % nbuf
            gh[j].wait()
            oh[j] = pltpu.async_copy(
                bufs[s], out_hbm.at[pl.ds(base + j * _CHUNK, _CHUNK)],
                osems[s])
            nxt = j + nbuf
            if nxt < n_chunks:
                oh[j].wait()
                gh[nxt] = pltpu.async_copy(
                    emb_hbm.at[idx_v.at[nxt]], bufs[s], gsems[s])
        for j in range(max(0, n_chunks - nbuf), n_chunks):
            oh[j].wait()

        if do_ctr:
            @pl.when(wid == 0)
            def _():
                pltpu.sync_copy(ctr_idx_hbm, ctr_idx_v)
                pltpu.async_copy(emb_hbm.at[ctr_idx_v], ctr_rows, sem_c).wait()
                pltpu.sync_copy(ctr_rows, ctr_out_hbm)

    if do_ctr:
        return gather_kernel(emb, idx3, center_idx)
    res = gather_kernel(emb, idx3)
    return res[0] if isinstance(res, (tuple, list)) else res


def _dg(a, b, dims):
    return lax.dot_general(a, b, (dims, ((), ())),
                           preferred_element_type=jnp.float32)


def _tc_mlp(gathered, ctr_rows, ef32, seg_cols,
            w1a2a, w1a2b, w1b2a, w1b2b, w1cab, b1m, w2two, b2r, p0):
    t_half, d = gathered.shape
    n_ctr = ctr_rows.shape[0]
    blk, nb_total = seg_cols.shape
    nb_h = t_half // blk
    nsteps = nb_h // 2

    def body(g_ref, c_ref, ef_ref, seg_ref,
             w1a2a_ref, w1a2b_ref, w1b2a_ref, w1b2b_ref,
             w1cab_ref, b1m_ref, w2two_ref, b2_ref, out_ref):
        i = pl.program_id(0)
        sel32 = (lax.broadcasted_iota(jnp.int32, (nb_total, 2 * n_ctr), 0) ==
                 p0 + 2 * i +
                 (lax.broadcasted_iota(jnp.int32, (nb_total, 2 * n_ctr), 1)
                  // n_ctr)).astype(jnp.float32)
        segsel = _dg(seg_ref[...], sel32, (((1,), (0,))))
        modrow = (lax.broadcasted_iota(jnp.int32, (1, 2 * n_ctr), 1)
                  % n_ctr).astype(jnp.float32)
        oh2 = (segsel == modrow).astype(jnp.float32)

        ctr = c_ref[...]
        cmab = jnp.concatenate([
            _dg(ctr, w1a2a_ref[...], (((1,), (0,)))),
            _dg(ctr, w1a2b_ref[...], (((1,), (0,)))),
        ], axis=0) + b1m_ref[...]

        g = g_ref[...]
        ef2 = ef_ref[0]
        pre = _dg(g[:blk], w1b2a_ref[...], (((1,), (0,))))
        pre += _dg(g[blk:], w1b2b_ref[...], (((1,), (0,))))
        pre += _dg(ef2, w1cab_ref[...], (((0,), (0,))))
        pre += _dg(oh2, cmab, (((1,), (0,))))
        h = jnp.maximum(pre, 0.0)
        pair = _dg(h, w2two_ref[...], (((1,), (0,))))
        scat16 = (lax.broadcasted_iota(jnp.int32, (2, nb_h), 1) ==
                  2 * i + lax.broadcasted_iota(jnp.int32, (2, nb_h), 0)
                  ).astype(jnp.float32)
        scat = _dg(pair, scat16, (((1,), (0,))))

        @pl.when(i == 0)
        def _():
            out_ref[...] = scat + b2_ref[...]

        @pl.when(i > 0)
        def _():
            out_ref[...] += scat

    pair0 = p0 // 2
    return pl.pallas_call(
        body,
        grid=(nsteps,),
        in_specs=[
            pl.BlockSpec((2 * blk, d), lambda i: (i, 0)),
            pl.BlockSpec((n_ctr, d), lambda i: (0, 0)),
            pl.BlockSpec((1, 2 * 16, blk), lambda i, o=pair0: (i + o, 0, 0)),
            pl.BlockSpec((blk, nb_total), lambda i: (0, 0)),
            pl.BlockSpec((d, 2 * 64), lambda i: (0, 0)),
            pl.BlockSpec((d, 2 * 64), lambda i: (0, 0)),
            pl.BlockSpec((d, 2 * 64), lambda i: (0, 0)),
            pl.BlockSpec((d, 2 * 64), lambda i: (0, 0)),
            pl.BlockSpec((2 * 16, 2 * 64), lambda i: (0, 0)),
            pl.BlockSpec((2 * n_ctr, 2 * 64), lambda i: (0, 0)),
            pl.BlockSpec((2 * 64, 2), lambda i: (0, 0)),
            pl.BlockSpec((1, 1), lambda i: (0, 0)),
        ],
        out_specs=pl.BlockSpec((blk, nb_h), lambda i: (0, 0)),
        out_shape=jax.ShapeDtypeStruct((blk, nb_h), jnp.float32),
    )(gathered, ctr_rows, ef32, seg_cols,
      w1a2a, w1a2b, w1b2a, w1b2b, w1cab, b1m, w2two, b2r)


def kernel(emb, center_idx, neighbor_idx, edge_feats, segment_ids,
           W1, b1, W2, b2):
    center_idx = center_idx.astype(jnp.int32)
    t_total = neighbor_idx.shape[0]
    d = emb.shape[1]
    d_mid = W1.shape[1]
    nb = t_total // _BLK
    t_half = t_total // _HALVES
    nb_h = nb // _HALVES

    w1a, w1b, w1c = W1[:d], W1[d:2 * d], W1[2 * d:]
    z = jnp.zeros_like
    pad_r = lambda w: jnp.concatenate([w, z(w)], axis=1)
    pad_l = lambda w: jnp.concatenate([z(w), w], axis=1)
    w2two = jnp.zeros((2 * d_mid, 2), jnp.float32)
    w2two = w2two.at[:d_mid, 0].set(W2[:, 0]).at[d_mid:, 1].set(W2[:, 0])
    w1cab = jnp.concatenate([pad_r(w1c), pad_l(w1c)], axis=0)
    b1a = jnp.concatenate([b1, jnp.zeros_like(b1)]).reshape(1, 2 * d_mid)
    b1b = jnp.concatenate([jnp.zeros_like(b1), b1]).reshape(1, 2 * d_mid)
    n_ctr = center_idx.shape[0]
    b1m = jnp.concatenate([jnp.broadcast_to(b1a, (n_ctr, 2 * d_mid)),
                           jnp.broadcast_to(b1b, (n_ctr, 2 * d_mid))], axis=0)
    seg_cols = segment_ids.reshape(nb, _BLK).T.astype(jnp.float32)
    b2r = b2.reshape(1, 1)

    ef_t = edge_feats.T
    d_e = edge_feats.shape[1]
    s_total = t_total // (2 * _BLK)
    ef32 = (ef_t.reshape(d_e, s_total, 2, _BLK)
            .transpose(1, 2, 0, 3).reshape(s_total, 2 * d_e, _BLK))

    gathered = []
    ctr_rows = None
    for h in range(_HALVES):
        sl = neighbor_idx[h * t_half:(h + 1) * t_half]
        if h == 0:
            g0, ctr_rows = _sc_gather(emb, sl, center_idx)
            gathered.append(g0)
        else:
            gathered.append(_sc_gather(emb, sl))

    outs = []
    for h in range(_HALVES):
        outs.append(_tc_mlp(
            gathered[h], ctr_rows, ef32, seg_cols,
            pad_r(w1a), pad_l(w1a), pad_r(w1b), pad_l(w1b),
            w1cab, b1m, w2two, b2r,
            p0=h * nb_h))
    out = outs[0] if _HALVES == 1 else jnp.concatenate(outs, axis=1)
    return out.T.reshape(t_total)
